# baseline (speedup 1.0000x reference)
import jax
import jax.numpy as jnp
from jax.experimental import pallas as pl
from jax.experimental.pallas import tpu as pltpu

_N = 8192
_H = 256
_B = 64

_BQ = 1024
_CK = 1024
_NB = _N // _BQ
_SUB = 128


def _body(x_ref, wemb_ref, bemb_ref, wq_ref, bq_ref, wk_ref, bk_ref,
          wv_ref, bv_ref, segrow_ref, segcol_ref, seg_smem,
          out_ref,
          q_s, k_s, v_s, qn2_s, att_scr, smax_ref, ssum_ref, cnt_ref,
          kmax_smem):
    i = pl.program_id(0)

    @pl.when(i == 0)
    def _():
        smax_ref[...] = jnp.full((_B, 1, _H), -jnp.inf, jnp.float32)
        ssum_ref[...] = jnp.zeros((_B, _H), jnp.float32)
        cnt_ref[...] = jnp.zeros((_B, 1), jnp.float32)
        kmax_smem[0] = 0.0

    @pl.when(i < _NB)
    def _():
        x = x_ref[...].astype(jnp.bfloat16)
        h = jnp.maximum(
            jnp.dot(x, wemb_ref[...].astype(jnp.bfloat16),
                    preferred_element_type=jnp.float32)
            + bemb_ref[...], 0.0).astype(jnp.bfloat16)
        scale = jnp.float32(1.4426950408889634) / jnp.sqrt(jnp.float32(_H))
        q = (jnp.dot(h, wq_ref[...].astype(jnp.bfloat16),
                     preferred_element_type=jnp.float32)
             + bq_ref[...]) * scale
        rows = pl.ds(i * _BQ, _BQ)
        q_s[rows, :] = q.astype(jnp.bfloat16)
        qn2_s[rows, :] = jnp.sum(q * q, axis=1, keepdims=True)
        k = (jnp.dot(h, wk_ref[...].astype(jnp.bfloat16),
                     preferred_element_type=jnp.float32)
             + bk_ref[...])
        k_s[rows, :] = k.astype(jnp.bfloat16)
        v_s[rows, :] = (jnp.dot(h, wv_ref[...].astype(jnp.bfloat16),
                                preferred_element_type=jnp.float32)
                        + bv_ref[...]).astype(jnp.bfloat16)
        kn2 = jnp.sum(k * k, axis=1, keepdims=True)
        kmax_smem[0] = jnp.maximum(kmax_smem[0], jnp.max(kn2))

    @pl.when(i >= _NB)
    def _():
        j = i - _NB
        rows = pl.ds(j * _BQ, _BQ)
        q = q_s[rows, :]
        bound = jnp.sqrt(qn2_s[rows, :] * kmax_smem[0])
        l = jnp.zeros((_BQ, 1), jnp.float32)
        acc = jnp.zeros((_BQ, _H), jnp.float32)
        for c in range(_N // _CK):
            k_c = k_s[c * _CK:(c + 1) * _CK, :]
            v_c = v_s[c * _CK:(c + 1) * _CK, :]
            s = jax.lax.dot_general(q, k_c, (((1,), (1,)), ((), ())),
                                    preferred_element_type=jnp.float32)
            p = jnp.exp2(s - bound)
            l = l + jnp.sum(p, axis=1, keepdims=True)
            acc = acc + jnp.dot(p.astype(jnp.bfloat16), v_c,
                                preferred_element_type=jnp.float32)
        att_scr[...] = acc / l

        att = att_scr[...]
        segrow = segrow_ref[0]
        segcol = segcol_ref[0]

        ids = jax.lax.broadcasted_iota(jnp.int32, (_B, _BQ), 0)
        mask = jnp.where(segrow == ids, 1.0, 0.0)
        ssum_ref[...] += jnp.dot(mask, att,
                                 preferred_element_type=jnp.float32)
        cnt_ref[...] += jnp.sum(mask, axis=1, keepdims=True)

        base = j * _BQ
        for t in range(_BQ // _SUB):
            att_t = att[t * _SUB:(t + 1) * _SUB, :]
            segc_t = segcol[t * _SUB:(t + 1) * _SUB, :]
            lo = seg_smem[base + t * _SUB]
            hi = seg_smem[base + t * _SUB + _SUB - 1]

            def _seg_body(b, carry, att_t=att_t, segc_t=segc_t):
                masked = jnp.where(segc_t == b, att_t, -jnp.inf)
                mx = jnp.max(masked, axis=0, keepdims=True)
                cur = smax_ref[pl.ds(b, 1), :, :]
                smax_ref[pl.ds(b, 1), :, :] = jnp.maximum(cur, mx[None])
                return carry

            jax.lax.fori_loop(lo, hi + 1, _seg_body, 0)

    @pl.when(i == 2 * _NB - 1)
    def _():
        cnt = cnt_ref[...]
        nonempty = cnt > 0.0
        mx = jnp.where(nonempty, smax_ref[:, 0, :], 0.0)
        mean = jnp.where(nonempty,
                         ssum_ref[...] / jnp.maximum(cnt, 1.0), 0.0)
        out_ref[...] = jnp.concatenate([mx, mean], axis=1)


def kernel(x, W_emb, b_emb, Wq, bq, Wk, bk, Wv, bv, batch):
    seg = batch.astype(jnp.int32)
    segrow = seg.reshape(_NB, 1, _BQ)
    segcol = seg.reshape(_NB, _BQ, 1)
    row_spec = pl.BlockSpec((_BQ, _H), lambda i: (jnp.minimum(i, _NB - 1), 0))
    w_spec = pl.BlockSpec((_H, _H), lambda i: (0, 0))
    b_spec = pl.BlockSpec((1, _H), lambda i: (0, 0))
    segb = lambda i: (jnp.maximum(i - _NB, 0), 0, 0)
    return pl.pallas_call(
        _body,
        grid=(2 * _NB,),
        in_specs=[
            row_spec, w_spec, b_spec, w_spec, b_spec, w_spec, b_spec,
            w_spec, b_spec,
            pl.BlockSpec((1, 1, _BQ), segb),
            pl.BlockSpec((1, _BQ, 1), segb),
            pl.BlockSpec(memory_space=pltpu.SMEM),
        ],
        out_specs=pl.BlockSpec((_B, 2 * _H), lambda i: (0, 0)),
        out_shape=jax.ShapeDtypeStruct((_B, 2 * _H), jnp.float32),
        scratch_shapes=[
            pltpu.VMEM((_N, _H), jnp.bfloat16),
            pltpu.VMEM((_N, _H), jnp.bfloat16),
            pltpu.VMEM((_N, _H), jnp.bfloat16),
            pltpu.VMEM((_N, 1), jnp.float32),
            pltpu.VMEM((_BQ, _H), jnp.float32),
            pltpu.VMEM((_B, 1, _H), jnp.float32),
            pltpu.VMEM((_B, _H), jnp.float32),
            pltpu.VMEM((_B, 1), jnp.float32),
            pltpu.SMEM((1,), jnp.float32),
        ],
        compiler_params=pltpu.CompilerParams(
            dimension_semantics=("arbitrary",),
            vmem_limit_bytes=56 * 1024 * 1024,
        ),
        name="sentence_readout",
    )(x, W_emb, b_emb.reshape(1, _H),
      Wq, bq.reshape(1, _H), Wk, bk.reshape(1, _H),
      Wv, bv.reshape(1, _H), segrow, segcol, seg)


# speedup vs baseline: 1.0555x; 1.0555x over previous
import jax
import jax.numpy as jnp
from jax.experimental import pallas as pl
from jax.experimental.pallas import tpu as pltpu

_N = 8192
_H = 256
_B = 64

_BQ = 2048
_CK = 512
_NB = _N // _BQ
_SUB = 128


def _body(x_ref, wemb_ref, bemb_ref, wq_ref, bq_ref, wk_ref, bk_ref,
          wv_ref, bv_ref, segrow_ref, segcol_ref, seg_smem,
          out_ref,
          q_s, k_s, v_s, qn2_s, att_scr, smax_ref, ssum_ref, cnt_ref,
          kmax_smem):
    i = pl.program_id(0)

    @pl.when(i == 0)
    def _():
        smax_ref[...] = jnp.full((_B, 1, _H), -jnp.inf, jnp.float32)
        ssum_ref[...] = jnp.zeros((_B, _H), jnp.float32)
        cnt_ref[...] = jnp.zeros((_B, 1), jnp.float32)
        kmax_smem[0] = 0.0

    @pl.when(i < _NB)
    def _():
        x = x_ref[...].astype(jnp.bfloat16)
        h = jnp.maximum(
            jnp.dot(x, wemb_ref[...].astype(jnp.bfloat16),
                    preferred_element_type=jnp.float32)
            + bemb_ref[...], 0.0).astype(jnp.bfloat16)
        scale = jnp.float32(1.4426950408889634) / jnp.sqrt(jnp.float32(_H))
        q = (jnp.dot(h, wq_ref[...].astype(jnp.bfloat16),
                     preferred_element_type=jnp.float32)
             + bq_ref[...]) * scale
        rows = pl.ds(i * _BQ, _BQ)
        q_s[rows, :] = q.astype(jnp.bfloat16)
        qn2_s[rows, :] = jnp.sum(q * q, axis=1, keepdims=True)
        k = (jnp.dot(h, wk_ref[...].astype(jnp.bfloat16),
                     preferred_element_type=jnp.float32)
             + bk_ref[...])
        k_s[rows, :] = k.astype(jnp.bfloat16)
        v_s[rows, :] = (jnp.dot(h, wv_ref[...].astype(jnp.bfloat16),
                                preferred_element_type=jnp.float32)
                        + bv_ref[...]).astype(jnp.bfloat16)
        kn2 = jnp.sum(k * k, axis=1, keepdims=True)
        kmax_smem[0] = jnp.maximum(kmax_smem[0], jnp.max(kn2))

    @pl.when(i >= _NB)
    def _():
        j = i - _NB
        rows = pl.ds(j * _BQ, _BQ)
        q = q_s[rows, :]
        bound = jnp.sqrt(qn2_s[rows, :] * kmax_smem[0])
        l = jnp.zeros((_BQ, 1), jnp.float32)
        acc = jnp.zeros((_BQ, _H), jnp.float32)
        for c in range(_N // _CK):
            k_c = k_s[c * _CK:(c + 1) * _CK, :]
            v_c = v_s[c * _CK:(c + 1) * _CK, :]
            s = jax.lax.dot_general(q, k_c, (((1,), (1,)), ((), ())),
                                    preferred_element_type=jnp.float32)
            p = jnp.exp2(s - bound)
            l = l + jnp.sum(p, axis=1, keepdims=True)
            acc = acc + jnp.dot(p.astype(jnp.bfloat16), v_c,
                                preferred_element_type=jnp.float32)
        att_scr[...] = acc / l

        att = att_scr[...]
        segrow = segrow_ref[0]
        segcol = segcol_ref[0]

        ids = jax.lax.broadcasted_iota(jnp.int32, (_B, _BQ), 0)
        mask = jnp.where(segrow == ids, 1.0, 0.0)
        ssum_ref[...] += jnp.dot(mask, att,
                                 preferred_element_type=jnp.float32)
        cnt_ref[...] += jnp.sum(mask, axis=1, keepdims=True)

        base = j * _BQ
        for t in range(_BQ // _SUB):
            att_t = att[t * _SUB:(t + 1) * _SUB, :]
            segc_t = segcol[t * _SUB:(t + 1) * _SUB, :]
            lo = seg_smem[base + t * _SUB]
            hi = seg_smem[base + t * _SUB + _SUB - 1]

            def _seg_body(b, carry, att_t=att_t, segc_t=segc_t):
                masked = jnp.where(segc_t == b, att_t, -jnp.inf)
                mx = jnp.max(masked, axis=0, keepdims=True)
                cur = smax_ref[pl.ds(b, 1), :, :]
                smax_ref[pl.ds(b, 1), :, :] = jnp.maximum(cur, mx[None])
                return carry

            jax.lax.fori_loop(lo, hi + 1, _seg_body, 0)

    @pl.when(i == 2 * _NB - 1)
    def _():
        cnt = cnt_ref[...]
        nonempty = cnt > 0.0
        mx = jnp.where(nonempty, smax_ref[:, 0, :], 0.0)
        mean = jnp.where(nonempty,
                         ssum_ref[...] / jnp.maximum(cnt, 1.0), 0.0)
        out_ref[...] = jnp.concatenate([mx, mean], axis=1)


def kernel(x, W_emb, b_emb, Wq, bq, Wk, bk, Wv, bv, batch):
    seg = batch.astype(jnp.int32)
    segrow = seg.reshape(_NB, 1, _BQ)
    segcol = seg.reshape(_NB, _BQ, 1)
    row_spec = pl.BlockSpec((_BQ, _H), lambda i: (jnp.minimum(i, _NB - 1), 0))
    w_spec = pl.BlockSpec((_H, _H), lambda i: (0, 0))
    b_spec = pl.BlockSpec((1, _H), lambda i: (0, 0))
    segb = lambda i: (jnp.maximum(i - _NB, 0), 0, 0)
    return pl.pallas_call(
        _body,
        grid=(2 * _NB,),
        in_specs=[
            row_spec, w_spec, b_spec, w_spec, b_spec, w_spec, b_spec,
            w_spec, b_spec,
            pl.BlockSpec((1, 1, _BQ), segb),
            pl.BlockSpec((1, _BQ, 1), segb),
            pl.BlockSpec(memory_space=pltpu.SMEM),
        ],
        out_specs=pl.BlockSpec((_B, 2 * _H), lambda i: (0, 0)),
        out_shape=jax.ShapeDtypeStruct((_B, 2 * _H), jnp.float32),
        scratch_shapes=[
            pltpu.VMEM((_N, _H), jnp.bfloat16),
            pltpu.VMEM((_N, _H), jnp.bfloat16),
            pltpu.VMEM((_N, _H), jnp.bfloat16),
            pltpu.VMEM((_N, 1), jnp.float32),
            pltpu.VMEM((_BQ, _H), jnp.float32),
            pltpu.VMEM((_B, 1, _H), jnp.float32),
            pltpu.VMEM((_B, _H), jnp.float32),
            pltpu.VMEM((_B, 1), jnp.float32),
            pltpu.SMEM((1,), jnp.float32),
        ],
        compiler_params=pltpu.CompilerParams(
            dimension_semantics=("arbitrary",),
            vmem_limit_bytes=56 * 1024 * 1024,
        ),
        name="sentence_readout",
    )(x, W_emb, b_emb.reshape(1, _H),
      Wq, bq.reshape(1, _H), Wk, bk.reshape(1, _H),
      Wv, bv.reshape(1, _H), segrow, segcol, seg)


# speedup vs baseline: 1.0559x; 1.0003x over previous
import jax
import jax.numpy as jnp
from jax.experimental import pallas as pl
from jax.experimental.pallas import tpu as pltpu

_N = 8192
_H = 256
_B = 64

_BQ = 2048
_CK = 512
_NB = _N // _BQ
_SUB = 128


def _body(x_ref, wemb_ref, bemb_ref, wq_ref, bq_ref, wk_ref, bk_ref,
          wv_ref, bv_ref, segrow_ref, segcol_ref, seg_smem,
          out_ref,
          q_s, k_s, v_s, qn2_s, att_scr, smax_ref, ssum_ref, cnt_ref,
          kmax_smem):
    i = pl.program_id(0)

    @pl.when(i == 0)
    def _():
        smax_ref[...] = jnp.full((_B, 1, _H), -jnp.inf, jnp.float32)
        ssum_ref[...] = jnp.zeros((_B, _H), jnp.float32)
        cnt_ref[...] = jnp.zeros((_B, 1), jnp.float32)
        kmax_smem[0] = 0.0

    @pl.when(i < _NB)
    def _():
        x = x_ref[...].astype(jnp.bfloat16)
        h = jnp.maximum(
            jnp.dot(x, wemb_ref[...].astype(jnp.bfloat16),
                    preferred_element_type=jnp.float32)
            + bemb_ref[...], 0.0).astype(jnp.bfloat16)
        scale = jnp.float32(1.4426950408889634) / jnp.sqrt(jnp.float32(_H))
        q = (jnp.dot(h, wq_ref[...].astype(jnp.bfloat16),
                     preferred_element_type=jnp.float32)
             + bq_ref[...]) * scale
        rows = pl.ds(i * _BQ, _BQ)
        q_s[rows, :] = q.astype(jnp.bfloat16)
        qn2_s[rows, :] = jnp.sum(q * q, axis=1, keepdims=True)
        k = (jnp.dot(h, wk_ref[...].astype(jnp.bfloat16),
                     preferred_element_type=jnp.float32)
             + bk_ref[...])
        k_s[rows, :] = k.astype(jnp.bfloat16)
        v_s[rows, :] = (jnp.dot(h, wv_ref[...].astype(jnp.bfloat16),
                                preferred_element_type=jnp.float32)
                        + bv_ref[...]).astype(jnp.bfloat16)
        kn2 = jnp.sum(k * k, axis=1, keepdims=True)
        kmax_smem[0] = jnp.maximum(kmax_smem[0], jnp.max(kn2))

    @pl.when(i >= _NB)
    def _():
        j = i - _NB
        rows = pl.ds(j * _BQ, _BQ)
        q = q_s[rows, :]
        bound = jnp.sqrt(qn2_s[rows, :] * kmax_smem[0])
        l = jnp.zeros((_BQ, 1), jnp.float32)
        acc = jnp.zeros((_BQ, _H), jnp.float32)
        for c in range(_N // _CK):
            k_c = k_s[c * _CK:(c + 1) * _CK, :]
            v_c = v_s[c * _CK:(c + 1) * _CK, :]
            s = jax.lax.dot_general(q, k_c, (((1,), (1,)), ((), ())),
                                    preferred_element_type=jnp.float32)
            p = jnp.exp2(s - bound)
            l = l + jnp.sum(p, axis=1, keepdims=True)
            acc = acc + jnp.dot(p.astype(jnp.bfloat16), v_c,
                                preferred_element_type=jnp.float32)
        att_scr[...] = acc / l

        att = att_scr[...]
        segrow = segrow_ref[0]
        segcol = segcol_ref[0]

        ids = jax.lax.broadcasted_iota(jnp.int32, (_B, _BQ), 0)
        mask = jnp.where(segrow == ids, 1.0, 0.0)
        ssum_ref[...] += jnp.dot(mask, att,
                                 preferred_element_type=jnp.float32)
        cnt_ref[...] += jnp.sum(mask, axis=1, keepdims=True)

        base = j * _BQ
        for t in range(_BQ // _SUB):
            att_t = att[t * _SUB:(t + 1) * _SUB, :]
            segc_t = segcol[t * _SUB:(t + 1) * _SUB, :]
            lo = seg_smem[base + t * _SUB]
            hi = seg_smem[base + t * _SUB + _SUB - 1]

            def _seg_body(b, carry, att_t=att_t, segc_t=segc_t):
                masked = jnp.where(segc_t == b, att_t, -jnp.inf)
                mx = jnp.max(masked, axis=0, keepdims=True)
                cur = smax_ref[pl.ds(b, 1), :, :]
                smax_ref[pl.ds(b, 1), :, :] = jnp.maximum(cur, mx[None])
                return carry

            jax.lax.fori_loop(lo, hi + 1, _seg_body, 0)

    @pl.when(i == 2 * _NB - 1)
    def _():
        cnt = cnt_ref[...]
        nonempty = cnt > 0.0
        mx = jnp.where(nonempty, smax_ref[:, 0, :], 0.0)
        mean = jnp.where(nonempty,
                         ssum_ref[...] / jnp.maximum(cnt, 1.0), 0.0)
        out_ref[...] = jnp.concatenate([mx, mean], axis=1)


def kernel(x, W_emb, b_emb, Wq, bq, Wk, bk, Wv, bv, batch):
    seg = batch.astype(jnp.int32)
    segrow = seg.reshape(_NB, 1, _BQ)
    segcol = seg.reshape(_NB, _BQ, 1)
    row_spec = pl.BlockSpec((_BQ, _H), lambda i: (jnp.minimum(i, _NB - 1), 0))
    w_spec = pl.BlockSpec((_H, _H), lambda i: (0, 0))
    b_spec = pl.BlockSpec((1, _H), lambda i: (0, 0))
    segb = lambda i: (jnp.maximum(i - _NB, 0), 0, 0)
    return pl.pallas_call(
        _body,
        grid=(2 * _NB,),
        in_specs=[
            row_spec, w_spec, b_spec, w_spec, b_spec, w_spec, b_spec,
            w_spec, b_spec,
            pl.BlockSpec((1, 1, _BQ), segb),
            pl.BlockSpec((1, _BQ, 1), segb),
            pl.BlockSpec(memory_space=pltpu.SMEM),
        ],
        out_specs=pl.BlockSpec((_B, 2 * _H), lambda i: (0, 0)),
        out_shape=jax.ShapeDtypeStruct((_B, 2 * _H), jnp.float32),
        scratch_shapes=[
            pltpu.VMEM((_N, _H), jnp.bfloat16),
            pltpu.VMEM((_N, _H), jnp.bfloat16),
            pltpu.VMEM((_N, _H), jnp.bfloat16),
            pltpu.VMEM((_N, 1), jnp.float32),
            pltpu.VMEM((_BQ, _H), jnp.float32),
            pltpu.VMEM((_B, 1, _H), jnp.float32),
            pltpu.VMEM((_B, _H), jnp.float32),
            pltpu.VMEM((_B, 1), jnp.float32),
            pltpu.SMEM((1,), jnp.float32),
        ],
        compiler_params=pltpu.CompilerParams(
            dimension_semantics=("arbitrary",),
            vmem_limit_bytes=56 * 1024 * 1024,
            flags={"XLA_TPU_STORE_TO_LOAD_FORWARDING_WINDOW": 12288},
        ),
        name="sentence_readout",
    )(x, W_emb, b_emb.reshape(1, _H),
      Wq, bq.reshape(1, _H), Wk, bk.reshape(1, _H),
      Wv, bv.reshape(1, _H), segrow, segcol, seg)
